# initial kernel scaffold (unmeasured)
import jax
import jax.numpy as jnp
from jax import lax
from jax.experimental import pallas as pl
from jax.experimental.pallas import tpu as pltpu

N_DEV = 8
COMM_DTYPE = jnp.bfloat16


def kernel(x, w_mat, scale_x, scale_w):
    m, _k_shard = x.shape
    _, n = w_mat.shape
    m_per = m // N_DEV

    xb = x.astype(COMM_DTYPE)
    wb = w_mat.astype(COMM_DTYPE)
    s = (scale_x * scale_w).astype(jnp.float32)

    def body(x_ref, w_ref, s_ref, out_ref, comm_ref, send_sems, recv_sems):
        my = lax.axis_index("i")
        left = lax.rem(my + N_DEV - 1, N_DEV)
        right = lax.rem(my + 1, N_DEV)

        barrier_sem = pltpu.get_barrier_semaphore()
        for nbr in (left, right):
            pl.semaphore_signal(
                barrier_sem, inc=1,
                device_id=(nbr,), device_id_type=pl.DeviceIdType.MESH,
            )
        pl.semaphore_wait(barrier_sem, 2)

        def partial_chunk(c):
            xa = x_ref[pl.ds(c * m_per, m_per), :]
            return lax.dot_general(
                xa, w_ref[...],
                (((1,), (0,)), ((), ())),
                preferred_element_type=jnp.float32,
            )

        c0 = lax.rem(my + N_DEV - 1, N_DEV)
        comm_ref[0, :, :] = partial_chunk(c0).astype(COMM_DTYPE)

        for h in range(N_DEV - 1):
            send_slot = h % 2
            recv_slot = (h + 1) % 2
            rdma = pltpu.make_async_remote_copy(
                src_ref=comm_ref.at[send_slot],
                dst_ref=comm_ref.at[recv_slot],
                send_sem=send_sems.at[send_slot],
                recv_sem=recv_sems.at[recv_slot],
                device_id=(right,),
                device_id_type=pl.DeviceIdType.MESH,
            )
            rdma.start()
            c = lax.rem(my + N_DEV - 2 - h, N_DEV)
            p_local = partial_chunk(c)
            rdma.wait()
            acc = p_local + comm_ref[recv_slot, :, :].astype(jnp.float32)
            if h < N_DEV - 2:
                comm_ref[recv_slot, :, :] = acc.astype(COMM_DTYPE)
            else:
                out_ref[...] = jnp.maximum(acc * s_ref[0], 0.0)

    return pl.pallas_call(
        body,
        out_shape=jax.ShapeDtypeStruct((m_per, n), jnp.float32),
        in_specs=[
            pl.BlockSpec(memory_space=pltpu.VMEM),
            pl.BlockSpec(memory_space=pltpu.VMEM),
            pl.BlockSpec(memory_space=pltpu.SMEM),
        ],
        out_specs=pl.BlockSpec(memory_space=pltpu.VMEM),
        scratch_shapes=[
            pltpu.VMEM((2, m_per, n), COMM_DTYPE),
            pltpu.SemaphoreType.DMA((2,)),
            pltpu.SemaphoreType.DMA((2,)),
        ],
        compiler_params=pltpu.CompilerParams(collective_id=0),
    )(xb, wb, s)


# baseline (device time: 723490 ns/iter reference)
import jax
import jax.numpy as jnp
from jax import lax
from jax.experimental import pallas as pl
from jax.experimental.pallas import tpu as pltpu

N_DEV = 8
COMM_DTYPE = jnp.bfloat16


def kernel(x, w_mat, scale_x, scale_w):
    m, _k_shard = x.shape
    _, n = w_mat.shape
    m_per = m // N_DEV

    xb = x.astype(COMM_DTYPE)
    wb = w_mat.astype(COMM_DTYPE)
    s = (scale_x * scale_w).astype(jnp.float32)

    def body(x_ref, w_ref, s_ref, out_ref, comm_ref, send_sems, recv_sems):
        my = lax.axis_index("i")
        left = lax.rem(my + N_DEV - 1, N_DEV)
        right = lax.rem(my + 1, N_DEV)

        barrier_sem = pltpu.get_barrier_semaphore()
        for nbr in (left, right):
            pl.semaphore_signal(
                barrier_sem, inc=1,
                device_id=(nbr,), device_id_type=pl.DeviceIdType.MESH,
            )
        pl.semaphore_wait(barrier_sem, 2)

        def partial_chunk(c):
            xa = x_ref[pl.ds(c * m_per, m_per), :]
            return lax.dot_general(
                xa, w_ref[...],
                (((1,), (0,)), ((), ())),
                preferred_element_type=jnp.float32,
            )

        c0 = lax.rem(my + N_DEV - 1, N_DEV)
        comm_ref[0, :, :] = partial_chunk(c0).astype(COMM_DTYPE)

        for h in range(N_DEV - 1):
            send_slot = h % 2
            recv_slot = (h + 1) % 2
            rdma = pltpu.make_async_remote_copy(
                src_ref=comm_ref.at[send_slot],
                dst_ref=comm_ref.at[recv_slot],
                send_sem=send_sems.at[send_slot],
                recv_sem=recv_sems.at[recv_slot],
                device_id=(right,),
                device_id_type=pl.DeviceIdType.MESH,
            )
            rdma.start()
            c = lax.rem(my + N_DEV - 2 - h, N_DEV)
            p_local = partial_chunk(c)
            rdma.wait()
            acc = p_local + comm_ref[recv_slot, :, :].astype(jnp.float32)
            if h < N_DEV - 2:
                comm_ref[recv_slot, :, :] = acc.astype(COMM_DTYPE)
            else:
                out_ref[...] = jnp.maximum(acc * s_ref[0], 0.0)

    return pl.pallas_call(
        body,
        out_shape=jax.ShapeDtypeStruct((m_per, n), jnp.float32),
        in_specs=[
            pl.BlockSpec(memory_space=pltpu.VMEM),
            pl.BlockSpec(memory_space=pltpu.VMEM),
            pl.BlockSpec(memory_space=pltpu.SMEM),
        ],
        out_specs=pl.BlockSpec(memory_space=pltpu.VMEM),
        scratch_shapes=[
            pltpu.VMEM((2, m_per, n), COMM_DTYPE),
            pltpu.SemaphoreType.DMA((2,)),
            pltpu.SemaphoreType.DMA((2,)),
        ],
        compiler_params=pltpu.CompilerParams(
            collective_id=0,
            vmem_limit_bytes=100 * 1024 * 1024,
        ),
    )(xb, wb, s)


# device time: 411466 ns/iter; 1.7583x vs baseline; 1.7583x over previous
import jax
import jax.numpy as jnp
from jax import lax
from jax.experimental import pallas as pl
from jax.experimental.pallas import tpu as pltpu

N_DEV = 8
COMM_DTYPE = jnp.bfloat16


def kernel(x, w_mat, scale_x, scale_w):
    m, _k_shard = x.shape
    _, n = w_mat.shape
    m_per = m // N_DEV
    n2 = n // 2

    xb = x.astype(COMM_DTYPE)
    wb = w_mat.astype(COMM_DTYPE)
    s = (scale_x * scale_w).astype(jnp.float32)

    def body(x_ref, w_ref, s_ref, out_ref,
             comm_a, comm_b, send_a, recv_a, send_b, recv_b):
        my = lax.axis_index("i")
        left = lax.rem(my + N_DEV - 1, N_DEV)
        right = lax.rem(my + 1, N_DEV)

        barrier_sem = pltpu.get_barrier_semaphore()
        for nbr in (left, right):
            pl.semaphore_signal(
                barrier_sem, inc=1,
                device_id=(nbr,), device_id_type=pl.DeviceIdType.MESH,
            )
        pl.semaphore_wait(barrier_sem, 2)

        def partial_a(c):
            xa = x_ref[pl.ds(c * m_per, m_per), :]
            return lax.dot_general(
                xa, w_ref[:, :n2], (((1,), (0,)), ((), ())),
                preferred_element_type=jnp.float32,
            )

        def partial_b(c):
            xa = x_ref[pl.ds(c * m_per, m_per), :]
            return lax.dot_general(
                xa, w_ref[:, n2:], (((1,), (0,)), ((), ())),
                preferred_element_type=jnp.float32,
            )

        comm_a[0, :, :] = partial_a(lax.rem(my + N_DEV - 1, N_DEV)).astype(COMM_DTYPE)
        comm_b[0, :, :] = partial_b(lax.rem(my + 1, N_DEV)).astype(COMM_DTYPE)

        for h in range(N_DEV - 1):
            ss = h % 2
            rs = (h + 1) % 2
            rdma_a = pltpu.make_async_remote_copy(
                src_ref=comm_a.at[ss], dst_ref=comm_a.at[rs],
                send_sem=send_a.at[ss], recv_sem=recv_a.at[rs],
                device_id=(right,), device_id_type=pl.DeviceIdType.MESH,
            )
            rdma_b = pltpu.make_async_remote_copy(
                src_ref=comm_b.at[ss], dst_ref=comm_b.at[rs],
                send_sem=send_b.at[ss], recv_sem=recv_b.at[rs],
                device_id=(left,), device_id_type=pl.DeviceIdType.MESH,
            )
            rdma_a.start()
            rdma_b.start()
            ca = lax.rem(my + N_DEV - 2 - h, N_DEV)
            cb = lax.rem(my + 2 + h, N_DEV)
            pa = partial_a(ca)
            pb = partial_b(cb)
            rdma_a.wait()
            rdma_b.wait()
            acc_a = pa + comm_a[rs, :, :].astype(jnp.float32)
            acc_b = pb + comm_b[rs, :, :].astype(jnp.float32)
            if h < N_DEV - 2:
                comm_a[rs, :, :] = acc_a.astype(COMM_DTYPE)
                comm_b[rs, :, :] = acc_b.astype(COMM_DTYPE)
            else:
                sc = s_ref[0]
                out_ref[:, :n2] = jnp.maximum(acc_a * sc, 0.0)
                out_ref[:, n2:] = jnp.maximum(acc_b * sc, 0.0)

    return pl.pallas_call(
        body,
        out_shape=jax.ShapeDtypeStruct((m_per, n), jnp.float32),
        in_specs=[
            pl.BlockSpec(memory_space=pltpu.VMEM),
            pl.BlockSpec(memory_space=pltpu.VMEM),
            pl.BlockSpec(memory_space=pltpu.SMEM),
        ],
        out_specs=pl.BlockSpec(memory_space=pltpu.VMEM),
        scratch_shapes=[
            pltpu.VMEM((2, m_per, n2), COMM_DTYPE),
            pltpu.VMEM((2, m_per, n2), COMM_DTYPE),
            pltpu.SemaphoreType.DMA((2,)),
            pltpu.SemaphoreType.DMA((2,)),
            pltpu.SemaphoreType.DMA((2,)),
            pltpu.SemaphoreType.DMA((2,)),
        ],
        compiler_params=pltpu.CompilerParams(
            collective_id=0,
            vmem_limit_bytes=100 * 1024 * 1024,
        ),
    )(xb, wb, s)


# device time: 366503 ns/iter; 1.9740x vs baseline; 1.1227x over previous
import jax
import jax.numpy as jnp
from jax import lax
from jax.experimental import pallas as pl
from jax.experimental.pallas import tpu as pltpu

N_DEV = 8
Q = 2
COMM_DTYPE = jnp.bfloat16


def kernel(x, w_mat, scale_x, scale_w):
    m, _k_shard = x.shape
    _, n = w_mat.shape
    m_per = m // N_DEV
    nq = n // (2 * Q)

    xb = x.astype(COMM_DTYPE)
    wb = w_mat.astype(COMM_DTYPE)
    s = (scale_x * scale_w).astype(jnp.float32)

    def body(x_ref, w_ref, s_ref, out_ref,
             comm_a, comm_b, send_a, recv_a, send_b, recv_b):
        my = lax.axis_index("i")
        left = lax.rem(my + N_DEV - 1, N_DEV)
        right = lax.rem(my + 1, N_DEV)

        def col(d, q):
            return (d * Q + q) * nq

        def partial(c, d, q):
            xa = x_ref[pl.ds(c * m_per, m_per), :]
            return lax.dot_general(
                xa, w_ref[:, col(d, q):col(d, q) + nq],
                (((1,), (0,)), ((), ())),
                preferred_element_type=jnp.float32,
            )

        def mk(h, d, q):
            ss = h % 2
            rs = (h + 1) % 2
            comm = comm_a if d == 0 else comm_b
            send = send_a if d == 0 else send_b
            recv = recv_a if d == 0 else recv_b
            tgt = right if d == 0 else left
            return pltpu.make_async_remote_copy(
                src_ref=comm.at[ss, q], dst_ref=comm.at[rs, q],
                send_sem=send.at[ss, q], recv_sem=recv.at[rs, q],
                device_id=(tgt,), device_id_type=pl.DeviceIdType.MESH,
            )

        c0 = [lax.rem(my + N_DEV - 1, N_DEV), lax.rem(my + 1, N_DEV)]
        for q in range(Q):
            comm_a[0, q, :, :] = partial(c0[0], 0, q).astype(COMM_DTYPE)
            comm_b[0, q, :, :] = partial(c0[1], 1, q).astype(COMM_DTYPE)

        barrier_sem = pltpu.get_barrier_semaphore()
        for nbr in (left, right):
            pl.semaphore_signal(
                barrier_sem, inc=1,
                device_id=(nbr,), device_id_type=pl.DeviceIdType.MESH,
            )
        pl.semaphore_wait(barrier_sem, 2)

        sends = {}
        for q in range(Q):
            for d in range(2):
                r = mk(0, d, q)
                r.start()
                sends[(0, d, q)] = r

        sc = s_ref[0]
        for h in range(N_DEV - 1):
            rs = (h + 1) % 2
            ch = [lax.rem(my + N_DEV - 2 - h, N_DEV),
                  lax.rem(my + 2 + h, N_DEV)]
            for q in range(Q):
                for d in range(2):
                    comm = comm_a if d == 0 else comm_b
                    p = partial(ch[d], d, q)
                    sends[(h, d, q)].wait_recv()
                    acc = p + comm[rs, q, :, :].astype(jnp.float32)
                    if h < N_DEV - 2:
                        comm[rs, q, :, :] = acc.astype(COMM_DTYPE)
                        if h >= 1:
                            sends[(h - 1, d, q)].wait_send()
                        r = mk(h + 1, d, q)
                        r.start()
                        sends[(h + 1, d, q)] = r
                    else:
                        out_ref[:, col(d, q):col(d, q) + nq] = jnp.maximum(
                            acc * sc, 0.0)

        for q in range(Q):
            for d in range(2):
                sends[(N_DEV - 3, d, q)].wait_send()
                sends[(N_DEV - 2, d, q)].wait_send()

    return pl.pallas_call(
        body,
        out_shape=jax.ShapeDtypeStruct((m_per, n), jnp.float32),
        in_specs=[
            pl.BlockSpec(memory_space=pltpu.VMEM),
            pl.BlockSpec(memory_space=pltpu.VMEM),
            pl.BlockSpec(memory_space=pltpu.SMEM),
        ],
        out_specs=pl.BlockSpec(memory_space=pltpu.VMEM),
        scratch_shapes=[
            pltpu.VMEM((2, Q, m_per, nq), COMM_DTYPE),
            pltpu.VMEM((2, Q, m_per, nq), COMM_DTYPE),
            pltpu.SemaphoreType.DMA((2, Q)),
            pltpu.SemaphoreType.DMA((2, Q)),
            pltpu.SemaphoreType.DMA((2, Q)),
            pltpu.SemaphoreType.DMA((2, Q)),
        ],
        compiler_params=pltpu.CompilerParams(
            collective_id=0,
            vmem_limit_bytes=100 * 1024 * 1024,
        ),
    )(xb, wb, s)


# device time: 361375 ns/iter; 2.0020x vs baseline; 1.0142x over previous
import jax
import jax.numpy as jnp
from jax import lax
from jax.experimental import pallas as pl
from jax.experimental.pallas import tpu as pltpu

N_DEV = 8
Q = 4
COMM_DTYPE = jnp.bfloat16


def kernel(x, w_mat, scale_x, scale_w):
    m, _k_shard = x.shape
    _, n = w_mat.shape
    m_per = m // N_DEV
    nq = n // (2 * Q)

    xb = x.astype(COMM_DTYPE)
    wb = w_mat.astype(COMM_DTYPE)
    s = (scale_x * scale_w).astype(jnp.float32)

    def body(x_ref, w_ref, s_ref, out_ref,
             comm_a, comm_b, send_a, recv_a, send_b, recv_b):
        my = lax.axis_index("i")
        left = lax.rem(my + N_DEV - 1, N_DEV)
        right = lax.rem(my + 1, N_DEV)

        def col(d, q):
            return (d * Q + q) * nq

        def partial(c, d, q):
            xa = x_ref[pl.ds(c * m_per, m_per), :]
            return lax.dot_general(
                xa, w_ref[:, col(d, q):col(d, q) + nq],
                (((1,), (0,)), ((), ())),
                preferred_element_type=jnp.float32,
            )

        def mk(h, d, q):
            ss = h % 2
            rs = (h + 1) % 2
            comm = comm_a if d == 0 else comm_b
            send = send_a if d == 0 else send_b
            recv = recv_a if d == 0 else recv_b
            tgt = right if d == 0 else left
            return pltpu.make_async_remote_copy(
                src_ref=comm.at[ss, q], dst_ref=comm.at[rs, q],
                send_sem=send.at[ss, q], recv_sem=recv.at[rs, q],
                device_id=(tgt,), device_id_type=pl.DeviceIdType.MESH,
            )

        barrier_sem = pltpu.get_barrier_semaphore()
        for nbr in (left, right):
            pl.semaphore_signal(
                barrier_sem, inc=1,
                device_id=(nbr,), device_id_type=pl.DeviceIdType.MESH,
            )
        pl.semaphore_wait(barrier_sem, 2)

        c0 = [lax.rem(my + N_DEV - 1, N_DEV), lax.rem(my + 1, N_DEV)]
        sends = {}
        for q in range(Q):
            for d in range(2):
                comm = comm_a if d == 0 else comm_b
                comm[0, q, :, :] = partial(c0[d], d, q).astype(COMM_DTYPE)
                r = mk(0, d, q)
                r.start()
                sends[(0, d, q)] = r

        sc = s_ref[0]
        for h in range(N_DEV - 1):
            rs = (h + 1) % 2
            ch = [lax.rem(my + N_DEV - 2 - h, N_DEV),
                  lax.rem(my + 2 + h, N_DEV)]
            for q in range(Q):
                for d in range(2):
                    comm = comm_a if d == 0 else comm_b
                    p = partial(ch[d], d, q)
                    sends[(h, d, q)].wait_recv()
                    acc = p + comm[rs, q, :, :].astype(jnp.float32)
                    if h < N_DEV - 2:
                        comm[rs, q, :, :] = acc.astype(COMM_DTYPE)
                        if h >= 1:
                            sends[(h - 1, d, q)].wait_send()
                        r = mk(h + 1, d, q)
                        r.start()
                        sends[(h + 1, d, q)] = r
                    else:
                        out_ref[:, col(d, q):col(d, q) + nq] = jnp.maximum(
                            acc * sc, 0.0)

        for q in range(Q):
            for d in range(2):
                sends[(N_DEV - 3, d, q)].wait_send()
                sends[(N_DEV - 2, d, q)].wait_send()

    return pl.pallas_call(
        body,
        out_shape=jax.ShapeDtypeStruct((m_per, n), jnp.float32),
        in_specs=[
            pl.BlockSpec(memory_space=pltpu.VMEM),
            pl.BlockSpec(memory_space=pltpu.VMEM),
            pl.BlockSpec(memory_space=pltpu.SMEM),
        ],
        out_specs=pl.BlockSpec(memory_space=pltpu.VMEM),
        scratch_shapes=[
            pltpu.VMEM((2, Q, m_per, nq), COMM_DTYPE),
            pltpu.VMEM((2, Q, m_per, nq), COMM_DTYPE),
            pltpu.SemaphoreType.DMA((2, Q)),
            pltpu.SemaphoreType.DMA((2, Q)),
            pltpu.SemaphoreType.DMA((2, Q)),
            pltpu.SemaphoreType.DMA((2, Q)),
        ],
        compiler_params=pltpu.CompilerParams(
            collective_id=0,
            vmem_limit_bytes=100 * 1024 * 1024,
        ),
    )(xb, wb, s)


# device time: 357864 ns/iter; 2.0217x vs baseline; 1.0098x over previous
import jax
import jax.numpy as jnp
from jax import lax
from jax.experimental import pallas as pl
from jax.experimental.pallas import tpu as pltpu

N_DEV = 8
Q = 4
COMM_DTYPE = jnp.bfloat16
IN_DTYPE = jnp.float8_e4m3fn


def kernel(x, w_mat, scale_x, scale_w):
    m, _k_shard = x.shape
    _, n = w_mat.shape
    m_per = m // N_DEV
    nq = n // (2 * Q)

    x8 = x.astype(IN_DTYPE)
    w8 = w_mat.astype(IN_DTYPE)
    s = (scale_x * scale_w).astype(jnp.float32)

    def body(x_ref, w_ref, s_ref, out_ref,
             comm_a, comm_b, send_a, recv_a, send_b, recv_b):
        my = lax.axis_index("i")
        left = lax.rem(my + N_DEV - 1, N_DEV)
        right = lax.rem(my + 1, N_DEV)

        def col(d, q):
            return (d * Q + q) * nq

        def partial(c, d, q):
            xa = x_ref[pl.ds(c * m_per, m_per), :]
            return lax.dot_general(
                xa, w_ref[:, col(d, q):col(d, q) + nq],
                (((1,), (0,)), ((), ())),
                preferred_element_type=jnp.float32,
            )

        def mk(h, d, q):
            ss = h % 2
            rs = (h + 1) % 2
            comm = comm_a if d == 0 else comm_b
            send = send_a if d == 0 else send_b
            recv = recv_a if d == 0 else recv_b
            tgt = right if d == 0 else left
            return pltpu.make_async_remote_copy(
                src_ref=comm.at[ss, q], dst_ref=comm.at[rs, q],
                send_sem=send.at[ss, q], recv_sem=recv.at[rs, q],
                device_id=(tgt,), device_id_type=pl.DeviceIdType.MESH,
            )

        barrier_sem = pltpu.get_barrier_semaphore()
        for nbr in (left, right):
            pl.semaphore_signal(
                barrier_sem, inc=1,
                device_id=(nbr,), device_id_type=pl.DeviceIdType.MESH,
            )
        pl.semaphore_wait(barrier_sem, 2)

        c0 = [lax.rem(my + N_DEV - 1, N_DEV), lax.rem(my + 1, N_DEV)]
        sends = {}
        for q in range(Q):
            for d in range(2):
                comm = comm_a if d == 0 else comm_b
                comm[0, q, :, :] = partial(c0[d], d, q).astype(COMM_DTYPE)
                r = mk(0, d, q)
                r.start()
                sends[(0, d, q)] = r

        sc = s_ref[0]
        for h in range(N_DEV - 1):
            rs = (h + 1) % 2
            ch = [lax.rem(my + N_DEV - 2 - h, N_DEV),
                  lax.rem(my + 2 + h, N_DEV)]
            for q in range(Q):
                for d in range(2):
                    comm = comm_a if d == 0 else comm_b
                    p = partial(ch[d], d, q)
                    sends[(h, d, q)].wait_recv()
                    acc = p + comm[rs, q, :, :].astype(jnp.float32)
                    if h < N_DEV - 2:
                        comm[rs, q, :, :] = acc.astype(COMM_DTYPE)
                        if h >= 1:
                            sends[(h - 1, d, q)].wait_send()
                        r = mk(h + 1, d, q)
                        r.start()
                        sends[(h + 1, d, q)] = r
                    else:
                        out_ref[:, col(d, q):col(d, q) + nq] = jnp.maximum(
                            acc * sc, 0.0)

        for q in range(Q):
            for d in range(2):
                sends[(N_DEV - 3, d, q)].wait_send()
                sends[(N_DEV - 2, d, q)].wait_send()

    return pl.pallas_call(
        body,
        out_shape=jax.ShapeDtypeStruct((m_per, n), jnp.float32),
        in_specs=[
            pl.BlockSpec(memory_space=pltpu.VMEM),
            pl.BlockSpec(memory_space=pltpu.VMEM),
            pl.BlockSpec(memory_space=pltpu.SMEM),
        ],
        out_specs=pl.BlockSpec(memory_space=pltpu.VMEM),
        scratch_shapes=[
            pltpu.VMEM((2, Q, m_per, nq), COMM_DTYPE),
            pltpu.VMEM((2, Q, m_per, nq), COMM_DTYPE),
            pltpu.SemaphoreType.DMA((2, Q)),
            pltpu.SemaphoreType.DMA((2, Q)),
            pltpu.SemaphoreType.DMA((2, Q)),
            pltpu.SemaphoreType.DMA((2, Q)),
        ],
        compiler_params=pltpu.CompilerParams(
            collective_id=0,
            vmem_limit_bytes=100 * 1024 * 1024,
        ),
    )(x8, w8, s)


# device time: 353885 ns/iter; 2.0444x vs baseline; 1.0112x over previous
import jax
import jax.numpy as jnp
from jax import lax
from jax.experimental import pallas as pl
from jax.experimental.pallas import tpu as pltpu

N_DEV = 8
Q = 4
COMM_DTYPE = jnp.bfloat16
IN_DTYPE = jnp.float8_e4m3fn


def kernel(x, w_mat, scale_x, scale_w):
    m, _k_shard = x.shape
    _, n = w_mat.shape
    m_per = m // N_DEV
    nq = n // (2 * Q)

    x8 = x.astype(IN_DTYPE)
    w8 = w_mat.astype(IN_DTYPE)

    def body(x_ref, w_ref, out_ref,
             comm_a, comm_b, send_a, recv_a, send_b, recv_b):
        my = lax.axis_index("i")
        left = lax.rem(my + N_DEV - 1, N_DEV)
        right = lax.rem(my + 1, N_DEV)

        def col(d, q):
            return (d * Q + q) * nq

        def partial(c, d, q):
            xa = x_ref[pl.ds(c * m_per, m_per), :]
            return lax.dot_general(
                xa, w_ref[:, col(d, q):col(d, q) + nq],
                (((1,), (0,)), ((), ())),
                preferred_element_type=jnp.float32,
            )

        def mk(h, d, q):
            ss = h % 2
            rs = (h + 1) % 2
            comm = comm_a if d == 0 else comm_b
            send = send_a if d == 0 else send_b
            recv = recv_a if d == 0 else recv_b
            tgt = right if d == 0 else left
            return pltpu.make_async_remote_copy(
                src_ref=comm.at[ss, q], dst_ref=comm.at[rs, q],
                send_sem=send.at[ss, q], recv_sem=recv.at[rs, q],
                device_id=(tgt,), device_id_type=pl.DeviceIdType.MESH,
            )

        barrier_sem = pltpu.get_barrier_semaphore()
        for nbr in (left, right):
            pl.semaphore_signal(
                barrier_sem, inc=1,
                device_id=(nbr,), device_id_type=pl.DeviceIdType.MESH,
            )
        pl.semaphore_wait(barrier_sem, 2)

        c0 = [lax.rem(my + N_DEV - 1, N_DEV), lax.rem(my + 1, N_DEV)]
        sends = {}
        for q in range(Q):
            for d in range(2):
                comm = comm_a if d == 0 else comm_b
                comm[0, q, :, :] = partial(c0[d], d, q).astype(COMM_DTYPE)
                r = mk(0, d, q)
                r.start()
                sends[(0, d, q)] = r

        for h in range(N_DEV - 1):
            rs = (h + 1) % 2
            ch = [lax.rem(my + N_DEV - 2 - h, N_DEV),
                  lax.rem(my + 2 + h, N_DEV)]
            for q in range(Q):
                for d in range(2):
                    comm = comm_a if d == 0 else comm_b
                    p = partial(ch[d], d, q)
                    sends[(h, d, q)].wait_recv()
                    acc = p + comm[rs, q, :, :].astype(jnp.float32)
                    if h < N_DEV - 2:
                        comm[rs, q, :, :] = acc.astype(COMM_DTYPE)
                        if h >= 1:
                            sends[(h - 1, d, q)].wait_send()
                        r = mk(h + 1, d, q)
                        r.start()
                        sends[(h + 1, d, q)] = r
                    else:
                        out_ref[:, col(d, q):col(d, q) + nq] = acc.astype(
                            COMM_DTYPE)

        for q in range(Q):
            for d in range(2):
                sends[(N_DEV - 3, d, q)].wait_send()
                sends[(N_DEV - 2, d, q)].wait_send()

    acc = pl.pallas_call(
        body,
        out_shape=jax.ShapeDtypeStruct((m_per, n), COMM_DTYPE),
        in_specs=[
            pl.BlockSpec(memory_space=pltpu.VMEM),
            pl.BlockSpec(memory_space=pltpu.VMEM),
        ],
        out_specs=pl.BlockSpec(memory_space=pltpu.VMEM),
        scratch_shapes=[
            pltpu.VMEM((2, Q, m_per, nq), COMM_DTYPE),
            pltpu.VMEM((2, Q, m_per, nq), COMM_DTYPE),
            pltpu.SemaphoreType.DMA((2, Q)),
            pltpu.SemaphoreType.DMA((2, Q)),
            pltpu.SemaphoreType.DMA((2, Q)),
            pltpu.SemaphoreType.DMA((2, Q)),
        ],
        compiler_params=pltpu.CompilerParams(
            collective_id=0,
            vmem_limit_bytes=100 * 1024 * 1024,
        ),
    )(x8, w8)

    s = (scale_x * scale_w).astype(jnp.float32)
    return jnp.maximum(acc.astype(jnp.float32) * s, 0.0)


# device time: 341454 ns/iter; 2.1189x vs baseline; 1.0364x over previous
import jax
import jax.numpy as jnp
from jax import lax
from jax.experimental import pallas as pl
from jax.experimental.pallas import tpu as pltpu

N_DEV = 8
Q = 4
COMM_DTYPE = jnp.bfloat16
IN_DTYPE = jnp.float8_e4m3fn


def kernel(x, w_mat, scale_x, scale_w):
    m, k_shard = x.shape
    _, n = w_mat.shape
    m_per = m // N_DEV
    nq = n // (2 * Q)
    nw = n // 4

    def body(x_ref, w_ref, out_ref,
             x8, w8, xst, wst,
             sem_x, sem_w,
             comm_a, comm_b, send_a, recv_a, send_b, recv_b):
        my = lax.axis_index("i")
        left = lax.rem(my + N_DEV - 1, N_DEV)
        right = lax.rem(my + 1, N_DEV)

        def col(d, q):
            return (d * Q + q) * nq

        def partial(c, d, q):
            xa = x8[pl.ds(c * m_per, m_per), :]
            return lax.dot_general(
                xa, w8[:, col(d, q):col(d, q) + nq],
                (((1,), (0,)), ((), ())),
                preferred_element_type=jnp.float32,
            )

        def mk(h, d, q):
            ss = h % 2
            rs = (h + 1) % 2
            comm = comm_a if d == 0 else comm_b
            send = send_a if d == 0 else send_b
            recv = recv_a if d == 0 else recv_b
            tgt = right if d == 0 else left
            return pltpu.make_async_remote_copy(
                src_ref=comm.at[ss, q], dst_ref=comm.at[rs, q],
                send_sem=send.at[ss, q], recv_sem=recv.at[rs, q],
                device_id=(tgt,), device_id_type=pl.DeviceIdType.MESH,
            )

        barrier_sem = pltpu.get_barrier_semaphore()
        for nbr in (left, right):
            pl.semaphore_signal(
                barrier_sem, inc=1,
                device_id=(nbr,), device_id_type=pl.DeviceIdType.MESH,
            )
        pl.semaphore_wait(barrier_sem, 2)

        cp_x = pltpu.make_async_copy(x_ref, xst, sem_x)
        cp_x.start()
        wblocks = [0, 2, 1, 3]

        def w_cp(i):
            b = wblocks[i]
            return pltpu.make_async_copy(
                w_ref.at[:, b * nw:(b + 1) * nw], wst.at[i % 2],
                sem_w.at[i % 2])

        wcps = [w_cp(i) for i in range(4)]
        wcps[0].start()
        wcps[1].start()
        cp_x.wait()
        x8[...] = xst[...].astype(IN_DTYPE)

        blk_subs = {0: (0, (0, 1)), 1: (0, (2, 3)),
                    2: (1, (0, 1)), 3: (1, (2, 3))}
        c0 = [lax.rem(my + N_DEV - 1, N_DEV), lax.rem(my + 1, N_DEV)]
        sends = {}
        for i in range(4):
            b = wblocks[i]
            wcps[i].wait()
            w8[:, b * nw:(b + 1) * nw] = wst[i % 2, :, :].astype(IN_DTYPE)
            if i + 2 < 4:
                wcps[i + 2].start()
            d, qs = blk_subs[b]
            comm = comm_a if d == 0 else comm_b
            for q in qs:
                comm[0, q, :, :] = partial(c0[d], d, q).astype(COMM_DTYPE)
                r = mk(0, d, q)
                r.start()
                sends[(0, d, q)] = r

        for h in range(N_DEV - 1):
            rs = (h + 1) % 2
            ch = [lax.rem(my + N_DEV - 2 - h, N_DEV),
                  lax.rem(my + 2 + h, N_DEV)]
            for q in range(Q):
                for d in range(2):
                    comm = comm_a if d == 0 else comm_b
                    p = partial(ch[d], d, q)
                    sends[(h, d, q)].wait_recv()
                    acc = p + comm[rs, q, :, :].astype(jnp.float32)
                    if h < N_DEV - 2:
                        comm[rs, q, :, :] = acc.astype(COMM_DTYPE)
                        if h >= 1:
                            sends[(h - 1, d, q)].wait_send()
                        r = mk(h + 1, d, q)
                        r.start()
                        sends[(h + 1, d, q)] = r
                    else:
                        out_ref[:, col(d, q):col(d, q) + nq] = acc.astype(
                            COMM_DTYPE)

        for q in range(Q):
            for d in range(2):
                sends[(N_DEV - 3, d, q)].wait_send()
                sends[(N_DEV - 2, d, q)].wait_send()

    acc = pl.pallas_call(
        body,
        out_shape=jax.ShapeDtypeStruct((m_per, n), COMM_DTYPE),
        in_specs=[
            pl.BlockSpec(memory_space=pltpu.HBM),
            pl.BlockSpec(memory_space=pltpu.HBM),
        ],
        out_specs=pl.BlockSpec(memory_space=pltpu.VMEM),
        scratch_shapes=[
            pltpu.VMEM((m, k_shard), IN_DTYPE),
            pltpu.VMEM((k_shard, n), IN_DTYPE),
            pltpu.VMEM((m, k_shard), jnp.float32),
            pltpu.VMEM((2, k_shard, nw), jnp.float32),
            pltpu.SemaphoreType.DMA,
            pltpu.SemaphoreType.DMA((2,)),
            pltpu.VMEM((2, Q, m_per, nq), COMM_DTYPE),
            pltpu.VMEM((2, Q, m_per, nq), COMM_DTYPE),
            pltpu.SemaphoreType.DMA((2, Q)),
            pltpu.SemaphoreType.DMA((2, Q)),
            pltpu.SemaphoreType.DMA((2, Q)),
            pltpu.SemaphoreType.DMA((2, Q)),
        ],
        compiler_params=pltpu.CompilerParams(
            collective_id=0,
            vmem_limit_bytes=100 * 1024 * 1024,
        ),
    )(x, w_mat)

    s = (scale_x * scale_w).astype(jnp.float32)
    return jnp.maximum(acc.astype(jnp.float32) * s, 0.0)
